# Initial kernel scaffold; baseline (speedup 1.0000x reference)
import jax
import jax.numpy as jnp
from jax.experimental import pallas as pl


def kernel(x, params):
    raise NotImplementedError("write your pallas kernel here")


# speedup vs baseline: 153.0620x; 153.0620x over previous
import functools

import jax
import jax.numpy as jnp
from jax import lax
from jax.experimental import pallas as pl
from jax.experimental.pallas import tpu as pltpu
from jax.experimental.pallas import tpu_sc as plsc

_NC, _NS, _L = 2, 16, 16
_NW = _NC * _NS
_EPS = 1e-5


def _fps(xyz_rc, npoint):
  B, _, R, C = xyz_rc.shape
  n = R * C

  def body(x_ref, nxyz_ref):
    xyz = x_ref[...]
    xs, ys, zs = xyz[:, 0], xyz[:, 1], xyz[:, 2]
    iota_n = (lax.broadcasted_iota(jnp.int32, (1, R, C), 1) * C
              + lax.broadcasted_iota(jnp.int32, (1, R, C), 2))
    iota_s = lax.broadcasted_iota(jnp.int32, (1, 1, npoint), 2)

    def step(i, carry):
      dists, far, acc = carry
      oh = (iota_n == far).astype(jnp.float32)
      cx = jnp.sum(xs * oh, axis=(1, 2), keepdims=True)
      cy = jnp.sum(ys * oh, axis=(1, 2), keepdims=True)
      cz = jnp.sum(zs * oh, axis=(1, 2), keepdims=True)
      cstack = jnp.concatenate([cx, cy, cz], axis=1)
      acc = acc + jnp.where(iota_s == i, cstack, 0.0)
      dx = xs - cx
      dy = ys - cy
      dz = zs - cz
      d = dx * dx + dy * dy + dz * dz
      dists = jnp.minimum(dists, d)
      m = jnp.max(dists, axis=(1, 2), keepdims=True)
      far = jnp.min(jnp.where(dists == m, iota_n, n), axis=(1, 2),
                    keepdims=True)
      return dists, far, acc

    dists0 = jnp.full((B, R, C), 1e10, jnp.float32)
    far0 = jnp.zeros((B, 1, 1), jnp.int32)
    acc0 = jnp.zeros((B, 3, npoint), jnp.float32)
    _, _, acc = lax.fori_loop(0, npoint, step, (dists0, far0, acc0))
    nxyz_ref[...] = acc

  return pl.pallas_call(
      body,
      out_shape=jax.ShapeDtypeStruct((B, 3, npoint), jnp.float32),
  )(xyz_rc)


def _layer(H, V, scale, shift, Wt, bout, Sb, act):
  has_v = V is not None
  B, S, ns, Cin = H.shape
  Cout = Wt.shape[1]

  def body(*refs):
    i, j = pl.program_id(0), pl.program_id(1)
    it = iter(refs)
    h_ref = next(it)
    v_ref = next(it) if has_v else None
    sc_ref = next(it) if act else None
    sh_ref = next(it) if act else None
    w_ref = next(it)
    b_ref = next(it)
    out_ref = next(it)
    st_ref = next(it)

    X = h_ref[0]
    if has_v:
      X = X - v_ref[0][:, None, :]
    X = X.reshape(Sb * ns, Cin)
    if act:
      X = jnp.maximum(X * sc_ref[...] + sh_ref[...], 0.0)
    Y = jnp.dot(X, w_ref[...], preferred_element_type=jnp.float32) + b_ref[...]
    out_ref[0] = Y.reshape(Sb, ns, Cout)

    @pl.when(jnp.logical_and(i == 0, j == 0))
    def _init():
      st_ref[...] = jnp.zeros_like(st_ref)

    st_ref[0:1, :] += jnp.sum(Y, axis=0, keepdims=True)
    st_ref[1:2, :] += jnp.sum(Y * Y, axis=0, keepdims=True)

  in_arrays = [H]
  in_specs = [pl.BlockSpec((1, Sb, ns, Cin), lambda i, j: (i, j, 0, 0))]
  if has_v:
    in_arrays.append(V)
    in_specs.append(pl.BlockSpec((1, Sb, Cin), lambda i, j: (i, j, 0)))
  if act:
    in_arrays += [scale, shift]
    in_specs += [pl.BlockSpec((1, Cin), lambda i, j: (0, 0)),
                 pl.BlockSpec((1, Cin), lambda i, j: (0, 0))]
  in_arrays += [Wt, bout]
  in_specs += [pl.BlockSpec((Cin, Cout), lambda i, j: (0, 0)),
               pl.BlockSpec((1, Cout), lambda i, j: (0, 0))]

  return pl.pallas_call(
      body,
      grid=(B, S // Sb),
      in_specs=in_specs,
      out_specs=[pl.BlockSpec((1, Sb, ns, Cout), lambda i, j: (i, j, 0, 0)),
                 pl.BlockSpec((8, Cout), lambda i, j: (0, 0))],
      out_shape=[jax.ShapeDtypeStruct((B, S, ns, Cout), jnp.float32),
                 jax.ShapeDtypeStruct((8, Cout), jnp.float32)],
  )(*in_arrays)


def _pool(H, scale, shift, Sb):
  B, S, ns, C = H.shape

  def body(h_ref, sc_ref, sh_ref, o_ref):
    A = jnp.maximum(h_ref[0] * sc_ref[...] + sh_ref[...], 0.0)
    o_ref[0] = jnp.max(A, axis=1)

  return pl.pallas_call(
      body,
      grid=(B, S // Sb),
      in_specs=[pl.BlockSpec((1, Sb, ns, C), lambda i, j: (i, j, 0, 0)),
                pl.BlockSpec((1, C), lambda i, j: (0, 0)),
                pl.BlockSpec((1, C), lambda i, j: (0, 0))],
      out_specs=pl.BlockSpec((1, Sb, C), lambda i, j: (i, j, 0)),
      out_shape=jax.ShapeDtypeStruct((B, S, C), jnp.float32),
  )(H, scale, shift)


def _ballquery(xh, yh, zh, nxh, nyh, nzh, ns, radius):
  B, N = xh.shape
  S = nxh.shape[1]
  rpw = (B * S) // _NW
  r2 = float(radius) * float(radius)
  nchunks = N // _L
  mesh = plsc.VectorSubcoreMesh(core_axis_name="c", subcore_axis_name="s",
                                num_cores=_NC, num_subcores=_NS)

  @functools.partial(
      pl.kernel,
      out_type=jax.ShapeDtypeStruct((B * S, ns), jnp.int32),
      mesh=mesh,
      compiler_params=pltpu.CompilerParams(use_tc_tiling_on_sc=False,
                                           needs_layout_passes=False),
      scratch_types=[pltpu.VMEM((N,), jnp.float32),
                     pltpu.VMEM((N,), jnp.float32),
                     pltpu.VMEM((N,), jnp.float32),
                     pltpu.VMEM((S,), jnp.float32),
                     pltpu.VMEM((S,), jnp.float32),
                     pltpu.VMEM((S,), jnp.float32),
                     pltpu.VMEM((ns,), jnp.int32)],
  )
  def kern(xh_r, yh_r, zh_r, nxh_r, nyh_r, nzh_r, out_r,
           xv, yv, zv, nxv, nyv, nzv, iv):
    wid = lax.axis_index("s") * _NC + lax.axis_index("c")
    row0 = wid * rpw
    b = row0 // S
    pltpu.sync_copy(xh_r.at[b], xv)
    pltpu.sync_copy(yh_r.at[b], yv)
    pltpu.sync_copy(zh_r.at[b], zv)
    pltpu.sync_copy(nxh_r.at[b], nxv)
    pltpu.sync_copy(nyh_r.at[b], nyv)
    pltpu.sync_copy(nzh_r.at[b], nzv)
    lanes = lax.iota(jnp.int32, _L)

    def row_body(rr, _):
      row = row0 + rr
      sidx = jnp.full((_L,), row - b * S, jnp.int32)
      cx = plsc.load_gather(nxv, [sidx])
      cy = plsc.load_gather(nyv, [sidx])
      cz = plsc.load_gather(nzv, [sidx])

      def chunk(ci, carry):
        cnt, first = carry
        base = ci * _L
        dx = cx - xv[pl.ds(base, _L)]
        dy = cy - yv[pl.ds(base, _L)]
        dz = cz - zv[pl.ds(base, _L)]
        d2 = dx * dx + dy * dy + dz * dz
        w = d2 < r2
        cs = plsc.cumsum(w.astype(jnp.int32)) + cnt
        keep = jnp.logical_and(w, cs <= ns)
        rank = jnp.clip(cs - 1, 0, ns - 1)
        nvec = lanes + (base + b * N)
        plsc.store_scatter(iv, [rank], nvec, mask=keep)
        first = jnp.minimum(
            first, jnp.min(jnp.where(w, nvec, jnp.int32(1 << 30))))
        return jnp.max(cs), first

      cnt, first = lax.fori_loop(0, nchunks, chunk,
                                 (jnp.int32(0), jnp.int32(1 << 30)))
      i0 = jnp.full((_L,), first, jnp.int32)

      def fillk(ki, _):
        kvec = lax.iota(jnp.int32, _L) + ki * _L
        plsc.store_scatter(iv, [kvec], i0, mask=kvec >= cnt)
        return 0

      lax.fori_loop(0, ns // _L, fillk, 0)
      pltpu.sync_copy(iv, out_r.at[row])
      return 0

    lax.fori_loop(0, rpw, row_body, 0)

  return kern(xh, yh, zh, nxh, nyh, nzh)


def _sc_gather(tab, idx):
  P = idx.shape[0]
  D = tab.shape[1]
  ppw = P // _NW
  ch = 128
  mesh = plsc.VectorSubcoreMesh(core_axis_name="c", subcore_axis_name="s",
                                num_cores=_NC, num_subcores=_NS)

  @functools.partial(
      pl.kernel,
      out_type=jax.ShapeDtypeStruct((P, D), jnp.float32),
      mesh=mesh,
      compiler_params=pltpu.CompilerParams(use_tc_tiling_on_sc=False),
      scratch_types=[pltpu.VMEM((ch,), jnp.int32),
                     pltpu.VMEM((ch, D), jnp.float32),
                     pltpu.SemaphoreType.DMA],
  )
  def kern(tab_r, idx_r, out_r, iv, rv, sem):
    wid = lax.axis_index("s") * _NC + lax.axis_index("c")
    base = wid * ppw

    def body(ci, _):
      off = base + ci * ch
      pltpu.sync_copy(idx_r.at[pl.ds(off, ch)], iv)
      pltpu.async_copy(tab_r.at[iv], rv, sem).wait()
      pltpu.sync_copy(rv, out_r.at[pl.ds(off, ch)])
      return 0

    lax.fori_loop(0, ppw // ch, body, 0)

  return kern(tab, idx)


def _pad_rows(w, rows):
  if w.shape[0] == rows:
    return w
  return jnp.concatenate(
      [w, jnp.zeros((rows - w.shape[0], w.shape[1]), w.dtype)], axis=0)


def _affine_from_stats(st, P, gamma, beta):
  m = st[0] / P
  var = st[1] / P - m * m
  A = gamma / jnp.sqrt(var + _EPS)
  sh = beta - m * A
  return A.reshape(1, -1), sh.reshape(1, -1)


def _msg_stage(xyzp, nxyzp, table, radii, nsamples, scale_params):
  B, _, N = xyzp.shape
  S = nxyzp.shape[2]
  Dpad = table.shape[2]
  tab = table.reshape(B * N, Dpad)
  Vfull = jnp.concatenate(
      [jnp.transpose(nxyzp, (0, 2, 1)),
       jnp.zeros((B, S, Dpad - 3), jnp.float32)], axis=-1)
  outs = []
  for radius, ns, layers in zip(radii, nsamples, scale_params):
    idx = _ballquery(xyzp[:, 0], xyzp[:, 1], xyzp[:, 2],
                     nxyzp[:, 0], nxyzp[:, 1], nxyzp[:, 2], ns, radius)
    G = _sc_gather(tab, idx.reshape(-1)).reshape(B, S, ns, Dpad)
    P = B * S * ns
    Sb = max(8, 512 // ns)
    Hcur, st = _layer(G, Vfull, None, None,
                      _pad_rows(layers[0]['W'].T, Dpad),
                      layers[0]['b'].reshape(1, -1), Sb, act=False)
    A, sh = _affine_from_stats(st, P, layers[0]['gamma'], layers[0]['beta'])
    for li in range(1, len(layers)):
      p = layers[li]
      Hcur, st = _layer(Hcur, None, A, sh, p['W'].T, p['b'].reshape(1, -1),
                        Sb, act=True)
      A, sh = _affine_from_stats(st, P, p['gamma'], p['beta'])
    outs.append(_pool(Hcur, A, sh, Sb))
  return jnp.concatenate(outs, axis=-1)


def kernel(x, params):
  B, N, _ = x.shape
  x = x.astype(jnp.float32)
  xt = jnp.transpose(x, (0, 2, 1))

  S1 = 512
  nxyz1p = _fps(xt.reshape(B, 3, N // 128, 128), S1)
  new_xyz1 = jnp.transpose(nxyz1p, (0, 2, 1))
  tab1 = jnp.concatenate([x, x, jnp.zeros((B, N, 10), jnp.float32)], axis=-1)
  feats1 = _msg_stage(xt, nxyz1p, tab1,
                      [0.1, 0.2, 0.4], [16, 32, 128], params[0])

  S2 = 128
  nxyz2p = _fps(nxyz1p.reshape(B, 3, S1 // 128, 128), S2)
  new_xyz2 = jnp.transpose(nxyz2p, (0, 2, 1))
  tab2 = jnp.concatenate(
      [new_xyz1, feats1, jnp.zeros((B, S1, 13), jnp.float32)],
      axis=-1)
  feats2 = _msg_stage(nxyz1p, nxyz2p, tab2,
                      [0.2, 0.4, 0.8], [32, 64, 128], params[1])

  lC = params[2][0]
  X0 = jnp.concatenate(
      [new_xyz2, feats2, jnp.zeros((B, S2, 5), jnp.float32)],
      axis=-1).reshape(B, 1, S2, 392)
  P3 = B * S2
  H, st = _layer(X0, None, None, None, _pad_rows(lC[0]['W'].T, 392),
                 lC[0]['b'].reshape(1, -1), 1, act=False)
  A, sh = _affine_from_stats(st, P3, lC[0]['gamma'], lC[0]['beta'])
  H, st = _layer(H, None, A, sh, lC[1]['W'].T, lC[1]['b'].reshape(1, -1),
                 1, act=True)
  A, sh = _affine_from_stats(st, P3, lC[1]['gamma'], lC[1]['beta'])
  out = _pool(H, A, sh, 1)
  return out.reshape(B, 512)


# speedup vs baseline: 187.6095x; 1.2257x over previous
import functools

import jax
import jax.numpy as jnp
from jax import lax
from jax.experimental import pallas as pl
from jax.experimental.pallas import tpu as pltpu
from jax.experimental.pallas import tpu_sc as plsc

_NC, _NS, _L = 2, 16, 16
_NW = _NC * _NS
_EPS = 1e-5


def _fps(xyz_rc, npoint):
  B, _, R, C = xyz_rc.shape
  n = R * C

  def body(x_ref, nxyz_ref):
    xyz = x_ref[...]
    xs, ys, zs = xyz[:, 0], xyz[:, 1], xyz[:, 2]
    iota_n = (lax.broadcasted_iota(jnp.int32, (1, R, C), 1) * C
              + lax.broadcasted_iota(jnp.int32, (1, R, C), 2))
    iota_s = lax.broadcasted_iota(jnp.int32, (1, 1, npoint), 2)

    def step(i, carry):
      dists, far, acc = carry
      oh = (iota_n == far).astype(jnp.float32)
      cx = jnp.sum(xs * oh, axis=(1, 2), keepdims=True)
      cy = jnp.sum(ys * oh, axis=(1, 2), keepdims=True)
      cz = jnp.sum(zs * oh, axis=(1, 2), keepdims=True)
      cstack = jnp.concatenate([cx, cy, cz], axis=1)
      acc = acc + jnp.where(iota_s == i, cstack, 0.0)
      dx = xs - cx
      dy = ys - cy
      dz = zs - cz
      d = dx * dx + dy * dy + dz * dz
      dists = jnp.minimum(dists, d)
      m = jnp.max(dists, axis=(1, 2), keepdims=True)
      far = jnp.min(jnp.where(dists == m, iota_n, n), axis=(1, 2),
                    keepdims=True)
      return dists, far, acc

    dists0 = jnp.full((B, R, C), 1e10, jnp.float32)
    far0 = jnp.zeros((B, 1, 1), jnp.int32)
    acc0 = jnp.zeros((B, 3, npoint), jnp.float32)
    _, _, acc = lax.fori_loop(0, npoint, step, (dists0, far0, acc0))
    nxyz_ref[...] = acc

  return pl.pallas_call(
      body,
      out_shape=jax.ShapeDtypeStruct((B, 3, npoint), jnp.float32),
  )(xyz_rc)


def _layer(H, V, scale, shift, Wt, bout, Sb, act):
  has_v = V is not None
  B, S, ns, Cin = H.shape
  Cout = Wt.shape[1]

  def body(*refs):
    i, j = pl.program_id(0), pl.program_id(1)
    it = iter(refs)
    h_ref = next(it)
    v_ref = next(it) if has_v else None
    sc_ref = next(it) if act else None
    sh_ref = next(it) if act else None
    w_ref = next(it)
    b_ref = next(it)
    out_ref = next(it)
    st_ref = next(it)

    X = h_ref[0]
    if has_v:
      X = X - v_ref[0][:, None, :]
    X = X.reshape(Sb * ns, Cin)
    if act:
      X = jnp.maximum(X * sc_ref[...] + sh_ref[...], 0.0)
    Y = jnp.dot(X, w_ref[...], preferred_element_type=jnp.float32) + b_ref[...]
    out_ref[0] = Y.reshape(Sb, ns, Cout)

    @pl.when(jnp.logical_and(i == 0, j == 0))
    def _init():
      st_ref[...] = jnp.zeros_like(st_ref)

    st_ref[0:1, :] += jnp.sum(Y, axis=0, keepdims=True)
    st_ref[1:2, :] += jnp.sum(Y * Y, axis=0, keepdims=True)

  in_arrays = [H]
  in_specs = [pl.BlockSpec((1, Sb, ns, Cin), lambda i, j: (i, j, 0, 0))]
  if has_v:
    in_arrays.append(V)
    in_specs.append(pl.BlockSpec((1, Sb, Cin), lambda i, j: (i, j, 0)))
  if act:
    in_arrays += [scale, shift]
    in_specs += [pl.BlockSpec((1, Cin), lambda i, j: (0, 0)),
                 pl.BlockSpec((1, Cin), lambda i, j: (0, 0))]
  in_arrays += [Wt, bout]
  in_specs += [pl.BlockSpec((Cin, Cout), lambda i, j: (0, 0)),
               pl.BlockSpec((1, Cout), lambda i, j: (0, 0))]

  return pl.pallas_call(
      body,
      grid=(B, S // Sb),
      in_specs=in_specs,
      out_specs=[pl.BlockSpec((1, Sb, ns, Cout), lambda i, j: (i, j, 0, 0)),
                 pl.BlockSpec((8, Cout), lambda i, j: (0, 0))],
      out_shape=[jax.ShapeDtypeStruct((B, S, ns, Cout), jnp.float32),
                 jax.ShapeDtypeStruct((8, Cout), jnp.float32)],
  )(*in_arrays)


def _pool(H, scale, shift, Sb):
  B, S, ns, C = H.shape

  def body(h_ref, sc_ref, sh_ref, o_ref):
    A = jnp.maximum(h_ref[0] * sc_ref[...] + sh_ref[...], 0.0)
    o_ref[0] = jnp.max(A, axis=1)

  return pl.pallas_call(
      body,
      grid=(B, S // Sb),
      in_specs=[pl.BlockSpec((1, Sb, ns, C), lambda i, j: (i, j, 0, 0)),
                pl.BlockSpec((1, C), lambda i, j: (0, 0)),
                pl.BlockSpec((1, C), lambda i, j: (0, 0))],
      out_specs=pl.BlockSpec((1, Sb, C), lambda i, j: (i, j, 0)),
      out_shape=jax.ShapeDtypeStruct((B, S, C), jnp.float32),
  )(H, scale, shift)


def _ballquery3(xh, yh, zh, nxh, nyh, nzh, nss, radii):
  B, N = xh.shape
  S = nxh.shape[1]
  rpw = (B * S) // _NW
  r2s = [float(r) * float(r) for r in radii]
  nchunks = N // _L
  mesh = plsc.VectorSubcoreMesh(core_axis_name="c", subcore_axis_name="s",
                                num_cores=_NC, num_subcores=_NS)

  @functools.partial(
      pl.kernel,
      out_type=tuple(jax.ShapeDtypeStruct((B * S, ns), jnp.int32)
                     for ns in nss),
      mesh=mesh,
      compiler_params=pltpu.CompilerParams(use_tc_tiling_on_sc=False,
                                           needs_layout_passes=False),
      scratch_types=[pltpu.VMEM((N,), jnp.float32),
                     pltpu.VMEM((N,), jnp.float32),
                     pltpu.VMEM((N,), jnp.float32),
                     pltpu.VMEM((S,), jnp.float32),
                     pltpu.VMEM((S,), jnp.float32),
                     pltpu.VMEM((S,), jnp.float32),
                     pltpu.VMEM((nss[0],), jnp.int32),
                     pltpu.VMEM((nss[1],), jnp.int32),
                     pltpu.VMEM((nss[2],), jnp.int32)],
  )
  def kern(xh_r, yh_r, zh_r, nxh_r, nyh_r, nzh_r,
           out0_r, out1_r, out2_r, xv, yv, zv, nxv, nyv, nzv,
           iv0, iv1, iv2):
    ivs = (iv0, iv1, iv2)
    outs = (out0_r, out1_r, out2_r)
    wid = lax.axis_index("s") * _NC + lax.axis_index("c")
    row0 = wid * rpw
    b = row0 // S
    pltpu.sync_copy(xh_r.at[b], xv)
    pltpu.sync_copy(yh_r.at[b], yv)
    pltpu.sync_copy(zh_r.at[b], zv)
    pltpu.sync_copy(nxh_r.at[b], nxv)
    pltpu.sync_copy(nyh_r.at[b], nyv)
    pltpu.sync_copy(nzh_r.at[b], nzv)
    lanes = lax.iota(jnp.int32, _L)
    big = jnp.full((_L,), 1 << 30, jnp.int32)
    zero = jnp.zeros((_L,), jnp.int32)

    def row_body(rr, _):
      row = row0 + rr
      sidx = jnp.full((_L,), row - b * S, jnp.int32)
      cx = plsc.load_gather(nxv, [sidx])
      cy = plsc.load_gather(nyv, [sidx])
      cz = plsc.load_gather(nzv, [sidx])

      def chunk(ci, carry):
        base = ci * _L
        dx = cx - xv[pl.ds(base, _L)]
        dy = cy - yv[pl.ds(base, _L)]
        dz = cz - zv[pl.ds(base, _L)]
        d2 = dx * dx + dy * dy + dz * dz
        nvec = lanes + (base + b * N)
        off = jnp.full((_L,), base + b * N, jnp.int32)
        new = []
        for t in range(3):
          cnt, first = carry[2 * t], carry[2 * t + 1]
          w = d2 < r2s[t]
          cs = plsc.cumsum(w.astype(jnp.int32)) + cnt
          keep = jnp.logical_and(w, cs <= nss[t])
          rank = jnp.clip(cs - 1, 0, nss[t] - 1)
          plsc.store_scatter(ivs[t], [rank], nvec, mask=keep)
          pc = plsc.all_reduce_population_count(w)
          f = plsc.all_reduce_ffs(w)
          first = jnp.where(
              jnp.logical_and(cnt == 0, pc > 0), off + f, first)
          new += [cnt + pc, first]
        return tuple(new)

      carry = lax.fori_loop(0, nchunks, chunk,
                            (zero, big, zero, big, zero, big))
      for t in range(3):
        cnt, first = carry[2 * t], carry[2 * t + 1]

        def fillk(ki, _, t=t, cnt=cnt, first=first):
          kvec = lax.iota(jnp.int32, _L) + ki * _L
          plsc.store_scatter(ivs[t], [kvec], first, mask=kvec >= cnt)
          return 0

        lax.fori_loop(0, nss[t] // _L, fillk, 0)
        pltpu.sync_copy(ivs[t], outs[t].at[row])
      return 0

    lax.fori_loop(0, rpw, row_body, 0)

  return kern(xh, yh, zh, nxh, nyh, nzh)


def _sc_gather(tab, idx):
  P = idx.shape[0]
  D = tab.shape[1]
  ppw = P // _NW
  ch = 128
  mesh = plsc.VectorSubcoreMesh(core_axis_name="c", subcore_axis_name="s",
                                num_cores=_NC, num_subcores=_NS)

  @functools.partial(
      pl.kernel,
      out_type=jax.ShapeDtypeStruct((P, D), jnp.float32),
      mesh=mesh,
      compiler_params=pltpu.CompilerParams(use_tc_tiling_on_sc=False),
      scratch_types=[pltpu.VMEM((ppw,), jnp.int32),
                     pltpu.VMEM((ch, D), jnp.float32),
                     pltpu.VMEM((ch, D), jnp.float32),
                     pltpu.SemaphoreType.DMA,
                     pltpu.SemaphoreType.DMA],
  )
  def kern(tab_r, idx_r, out_r, iv, rv0, rv1, sem0, sem1):
    wid = lax.axis_index("s") * _NC + lax.axis_index("c")
    base = wid * ppw
    pltpu.sync_copy(idx_r.at[pl.ds(base, ppw)], iv)

    def body(g, _):
      c0 = 2 * g * ch
      c1 = c0 + ch
      h0 = pltpu.async_copy(tab_r.at[iv.at[pl.ds(c0, ch)]], rv0, sem0)
      h1 = pltpu.async_copy(tab_r.at[iv.at[pl.ds(c1, ch)]], rv1, sem1)
      h0.wait()
      pltpu.sync_copy(rv0, out_r.at[pl.ds(base + c0, ch)])
      h1.wait()
      pltpu.sync_copy(rv1, out_r.at[pl.ds(base + c1, ch)])
      return 0

    lax.fori_loop(0, ppw // (2 * ch), body, 0)

  return kern(tab, idx)


def _pad_rows(w, rows):
  if w.shape[0] == rows:
    return w
  return jnp.concatenate(
      [w, jnp.zeros((rows - w.shape[0], w.shape[1]), w.dtype)], axis=0)


def _affine_from_stats(st, P, gamma, beta):
  m = st[0] / P
  var = st[1] / P - m * m
  A = gamma / jnp.sqrt(var + _EPS)
  sh = beta - m * A
  return A.reshape(1, -1), sh.reshape(1, -1)


def _msg_stage(xyzp, nxyzp, table, radii, nsamples, scale_params):
  B, _, N = xyzp.shape
  S = nxyzp.shape[2]
  Dpad = table.shape[2]
  tab = table.reshape(B * N, Dpad)
  Vfull = jnp.concatenate(
      [jnp.transpose(nxyzp, (0, 2, 1)),
       jnp.zeros((B, S, Dpad - 3), jnp.float32)], axis=-1)
  idxs = _ballquery3(xyzp[:, 0], xyzp[:, 1], xyzp[:, 2],
                     nxyzp[:, 0], nxyzp[:, 1], nxyzp[:, 2], nsamples, radii)
  outs = []
  for idx, ns, layers in zip(idxs, nsamples, scale_params):
    G = _sc_gather(tab, idx.reshape(-1)).reshape(B, S, ns, Dpad)
    P = B * S * ns
    Sb = max(8, 512 // ns)
    Hcur, st = _layer(G, Vfull, None, None,
                      _pad_rows(layers[0]['W'].T, Dpad),
                      layers[0]['b'].reshape(1, -1), Sb, act=False)
    A, sh = _affine_from_stats(st, P, layers[0]['gamma'], layers[0]['beta'])
    for li in range(1, len(layers)):
      p = layers[li]
      Hcur, st = _layer(Hcur, None, A, sh, p['W'].T, p['b'].reshape(1, -1),
                        Sb, act=True)
      A, sh = _affine_from_stats(st, P, p['gamma'], p['beta'])
    outs.append(_pool(Hcur, A, sh, Sb))
  return jnp.concatenate(outs, axis=-1)


def kernel(x, params):
  B, N, _ = x.shape
  x = x.astype(jnp.float32)
  xt = jnp.transpose(x, (0, 2, 1))

  S1 = 512
  nxyz1p = _fps(xt.reshape(B, 3, N // 128, 128), S1)
  new_xyz1 = jnp.transpose(nxyz1p, (0, 2, 1))
  tab1 = jnp.concatenate([x, x, jnp.zeros((B, N, 10), jnp.float32)], axis=-1)
  feats1 = _msg_stage(xt, nxyz1p, tab1,
                      [0.1, 0.2, 0.4], [16, 32, 128], params[0])

  S2 = 128
  nxyz2p = _fps(nxyz1p.reshape(B, 3, S1 // 128, 128), S2)
  new_xyz2 = jnp.transpose(nxyz2p, (0, 2, 1))
  tab2 = jnp.concatenate(
      [new_xyz1, feats1, jnp.zeros((B, S1, 13), jnp.float32)],
      axis=-1)
  feats2 = _msg_stage(nxyz1p, nxyz2p, tab2,
                      [0.2, 0.4, 0.8], [32, 64, 128], params[1])

  lC = params[2][0]
  X0 = jnp.concatenate(
      [new_xyz2, feats2, jnp.zeros((B, S2, 5), jnp.float32)],
      axis=-1).reshape(B, 1, S2, 392)
  P3 = B * S2
  H, st = _layer(X0, None, None, None, _pad_rows(lC[0]['W'].T, 392),
                 lC[0]['b'].reshape(1, -1), 1, act=False)
  A, sh = _affine_from_stats(st, P3, lC[0]['gamma'], lC[0]['beta'])
  H, st = _layer(H, None, A, sh, lC[1]['W'].T, lC[1]['b'].reshape(1, -1),
                 1, act=True)
  A, sh = _affine_from_stats(st, P3, lC[1]['gamma'], lC[1]['beta'])
  out = _pool(H, A, sh, 1)
  return out.reshape(B, 512)
